# Initial kernel scaffold; baseline (speedup 1.0000x reference)
import jax
import jax.numpy as jnp
from jax.experimental import pallas as pl


def kernel(inputs, bn_gamma, bn_beta, w_q, keys, key_in, key_out):
    raise NotImplementedError("write your pallas kernel here")


# speedup vs baseline: 6.2862x; 6.2862x over previous
import functools

import jax
import jax.numpy as jnp
from jax import lax
from jax.experimental import pallas as pl
from jax.experimental.pallas import tpu as pltpu
from jax.experimental.pallas import tpu_sc as plsc

B, N, D = 2, 2048, 1024
T = B * N
H = 8
KD = 128
K = 4
NUM_KEYS = 128
NUM_EXPERTS = 16384
HK = H * K
BN_EPS = 1e-5


_STATS_BLK = 256


def _stats_body(x_ref, o_ref):
    j = pl.program_id(0)

    xb = x_ref[...]

    @pl.when(j == 0)
    def _sum():
        o_ref[...] = jnp.zeros_like(o_ref)
        o_ref[0:1, :] = jnp.sum(xb, axis=0)[None, :]

    @pl.when(j == 1)
    def _sqdev():
        mean = o_ref[0:1, :] * (1.0 / float(T))
        d = xb - mean
        o_ref[1:2, :] = jnp.sum(d * d, axis=0)[None, :]


_stats_call = pl.pallas_call(
    _stats_body,
    grid=(2,),
    in_specs=[pl.BlockSpec((T, D), lambda j: (0, 0))],
    out_specs=pl.BlockSpec((8, D), lambda j: (0, 0)),
    out_shape=jax.ShapeDtypeStruct((8, D), jnp.float32),
    compiler_params=pltpu.CompilerParams(
        dimension_semantics=("arbitrary",)),
)


_TB = 512


def _topk4(s, payload):
    m_cols = s.shape[1]
    iota = lax.broadcasted_iota(jnp.int32, s.shape, 1)
    cur = s
    ss, pp = [], []
    for _ in range(K):
        m = jnp.max(cur, axis=1, keepdims=True)
        pos = jnp.min(jnp.where(cur == m, iota, m_cols), axis=1, keepdims=True)
        sel = iota == pos
        ss.append(m)
        pp.append(jnp.sum(jnp.where(sel, payload, 0), axis=1, keepdims=True))
        cur = jnp.where(sel, -jnp.inf, cur)
    return jnp.concatenate(ss, axis=1), jnp.concatenate(pp, axis=1)


def _route_body(x_ref, stats_ref, g_ref, b_ref, wq_ref, keys_ref,
                idx_ref, gate_ref):
    x = x_ref[...]
    inv_cnt = 1.0 / float(T)
    mean = stats_ref[0:1, :] * inv_cnt
    var = stats_ref[1:2, :] * inv_cnt
    xn = (x - mean) / jnp.sqrt(var + BN_EPS) * g_ref[...] + b_ref[...]
    q = lax.dot_general(xn, wq_ref[...], (((1,), (0,)), ((), ())),
                        precision=lax.Precision.DEFAULT,
                        preferred_element_type=jnp.float32)
    q = q.astype(jnp.bfloat16)
    idx_cols, gate_cols = [], []
    for h in range(H):
        sims = []
        for p in range(2):
            off = (p * H + h) * KD
            qs = q[:, off:off + KD]
            km = keys_ref[p, h].astype(jnp.bfloat16)
            sims.append(lax.dot_general(
                qs, km, (((1,), (0,)), ((), ())),
                precision=lax.Precision.DEFAULT,
                preferred_element_type=jnp.float32))
        iota_k = lax.broadcasted_iota(jnp.int32, sims[0].shape, 1)
        sx, ix = _topk4(sims[0], iota_k)
        sy, iy = _topk4(sims[1], iota_k)
        cols_s, cols_i = [], []
        for a in range(K):
            for c in range(K):
                cols_s.append(sx[:, a:a + 1] + sy[:, c:c + 1])
                cols_i.append(ix[:, a:a + 1] * NUM_KEYS + iy[:, c:c + 1])
        s16 = jnp.concatenate(cols_s, axis=1)
        i16 = jnp.concatenate(cols_i, axis=1)
        sc, ei = _topk4(s16, i16)
        m = jnp.max(sc, axis=1, keepdims=True)
        e = jnp.exp(sc - m)
        gate_cols.append(e / jnp.sum(e, axis=1, keepdims=True))
        idx_cols.append(ei)
    idx_ref[...] = jnp.concatenate(idx_cols, axis=1)
    gate_ref[...] = jnp.concatenate(gate_cols, axis=1)


_route_call = pl.pallas_call(
    _route_body,
    grid=(T // _TB,),
    in_specs=[
        pl.BlockSpec((_TB, D), lambda i: (i, 0)),
        pl.BlockSpec((8, D), lambda i: (0, 0)),
        pl.BlockSpec((1, D), lambda i: (0, 0)),
        pl.BlockSpec((1, D), lambda i: (0, 0)),
        pl.BlockSpec((D, 2 * H * KD), lambda i: (0, 0)),
        pl.BlockSpec((2, H, KD, NUM_KEYS), lambda i: (0, 0, 0, 0)),
    ],
    out_specs=[
        pl.BlockSpec((_TB, HK), lambda i: (i, 0)),
        pl.BlockSpec((_TB, HK), lambda i: (i, 0)),
    ],
    out_shape=[
        jax.ShapeDtypeStruct((T, HK), jnp.int32),
        jax.ShapeDtypeStruct((T, HK), jnp.float32),
    ],
)


_NC, _NS = 2, 16
_NW = _NC * _NS
_TPW = T // _NW
_G = 8
_UPG = 2 * _G
_L = 16


def _sc_body(x_hbm, idx_hbm, gate_hbm, kin_hbm, kout_hbm, out_hbm,
             xg, idxg, gateg, rdn, rup, outg, sem_dn, sem_up):
    wid = lax.axis_index("s") * _NC + lax.axis_index("c")
    t0 = wid * _TPW
    jiota = lax.iota(jnp.int32, _L)

    def group_body(g, carry):
        tg = t0 + g * _G
        pltpu.sync_copy(x_hbm.at[pl.ds(tg, _G)], xg)
        pltpu.sync_copy(idx_hbm.at[pl.ds(2 * tg, _UPG)], idxg)
        pltpu.sync_copy(gate_hbm.at[pl.ds(2 * tg, _UPG)], gateg)

        def unit_body(u, c2):
            t_loc = u // 2
            half = u - 2 * t_loc
            cp_dn = pltpu.async_copy(kin_hbm.at[idxg.at[u]], rdn, sem_dn)
            cp_up = pltpu.async_copy(kout_hbm.at[idxg.at[u]], rup, sem_up)
            cp_dn.wait()
            xrow = xg.at[t_loc]

            def dot_step(c, accs):
                base = c * _L
                xv = xrow[pl.ds(base, _L)]
                new = list(accs)
                for j in range(_L):
                    rv = rdn[j, pl.ds(base, _L)]
                    new[j] = new[j] + rv * xv
                return tuple(new)

            accs = lax.fori_loop(
                0, D // _L, dot_step,
                tuple(jnp.zeros((_L,), jnp.float32) for _ in range(_L)))
            dots = jnp.zeros((_L,), jnp.float32)
            for j in range(_L):
                pal = accs[j] + lax.rev(accs[j], (0,))
                dj = (((pal[0] + pal[1]) + (pal[2] + pal[3]))
                      + ((pal[4] + pal[5]) + (pal[6] + pal[7])))
                dots = jnp.where(jiota == j, dj, dots)
            y = dots * 0.7071067811865476
            ay = jnp.abs(y)
            tt = 1.0 / (1.0 + 0.3275911 * ay)
            poly = ((((1.061405429 * tt - 1.453152027) * tt + 1.421413741)
                     * tt - 0.284496736) * tt + 0.254829592) * tt
            erf_a = 1.0 - poly * jnp.exp(-(ay * ay))
            erf_y = jnp.where(y >= 0.0, erf_a, -erf_a)
            gelu = 0.5 * dots * (1.0 + erf_y)
            w = gateg.at[u][...] * gelu
            cp_up.wait()
            ws = [w[j] for j in range(_L)]
            orow = outg.at[t_loc]
            is_first = half == 0

            def up_step(c, c3):
                base = c * _L
                terms = [ws[j] * rup[j, pl.ds(base, _L)] for j in range(_L)]
                while len(terms) > 1:
                    terms = [terms[i] + terms[i + 1]
                             for i in range(0, len(terms), 2)]
                prev = orow[pl.ds(base, _L)]
                orow[pl.ds(base, _L)] = jnp.where(is_first, terms[0],
                                                  prev + terms[0])
                return c3

            lax.fori_loop(0, D // _L, up_step, 0)
            return c2

        lax.fori_loop(0, _UPG, unit_body, 0)
        pltpu.sync_copy(outg, out_hbm.at[pl.ds(tg, _G)])
        return carry

    lax.fori_loop(0, _TPW // _G, group_body, 0)


def _make_sc_call():
    return functools.partial(
        pl.kernel,
        out_type=jax.ShapeDtypeStruct((T, D), jnp.float32),
        mesh=plsc.VectorSubcoreMesh(core_axis_name="c", subcore_axis_name="s"),
        compiler_params=pltpu.CompilerParams(use_tc_tiling_on_sc=False),
        scratch_types=[
            pltpu.VMEM((_G, D), jnp.float32),
            pltpu.VMEM((_UPG, _L), jnp.int32),
            pltpu.VMEM((_UPG, _L), jnp.float32),
            pltpu.VMEM((_L, D), jnp.float32),
            pltpu.VMEM((_L, D), jnp.float32),
            pltpu.VMEM((_G, D), jnp.float32),
            pltpu.SemaphoreType.DMA,
            pltpu.SemaphoreType.DMA,
        ],
    )(_sc_body)


_DIAG_JNP_TAIL = False
_DIAG_STATS_ONLY = False


def kernel(inputs, bn_gamma, bn_beta, w_q, keys, key_in, key_out):
    x = inputs.reshape(T, D)
    stats = _stats_call(x)
    if _DIAG_STATS_ONLY:
        b, n, d = inputs.shape
        mean = stats[0, :] * (1.0 / float(T))
        var = stats[1, :] * (1.0 / float(T))
        xs = (x - mean) / jnp.sqrt(var + BN_EPS) * bn_gamma + bn_beta
        xs = xs.reshape(b, n, d)
        q = xs @ w_q
        q = q.reshape(b, n, 2, H, KD)
        q = jnp.transpose(q, (2, 0, 1, 3, 4))
        sim = jnp.einsum('pbnhd,hkpd->pbnhk', q, keys)
        scores_x, indices_x = jax.lax.top_k(sim[0], K)
        scores_y, indices_y = jax.lax.top_k(sim[1], K)
        all_scores = scores_x[..., :, None] + scores_y[..., None, :]
        all_indices = indices_x[..., :, None] * NUM_KEYS + indices_y[..., None, :]
        all_scores = all_scores.reshape(*all_scores.shape[:-2], K * K)
        all_indices = all_indices.reshape(*all_indices.shape[:-2], K * K)
        scores, pk_indices = jax.lax.top_k(all_scores, K)
        indices = jnp.take_along_axis(all_indices, pk_indices, axis=-1)
        weights_down = key_in[indices]
        weights_up = key_out[indices]
        outputs = jnp.einsum('bnd,bnhkd->bnhk', inputs, weights_down)
        outputs = jax.nn.gelu(outputs, approximate=False)
        outputs = jax.nn.softmax(scores, axis=-1) * outputs
        outputs = jnp.einsum('bnhk,bnhkd->bnd', outputs, weights_up)
        return outputs
    keys_r = jnp.transpose(keys, (2, 0, 3, 1))
    idx, gate = _route_call(x, stats, bn_gamma.reshape(1, D),
                            bn_beta.reshape(1, D),
                            w_q.astype(jnp.bfloat16), keys_r)
    idx2 = idx.reshape(2 * T, _L)
    gate2 = gate.reshape(2 * T, _L)
    if _DIAG_JNP_TAIL:
        rows_dn = key_in[idx]
        rows_up = key_out[idx]
        dots = jnp.einsum('td,tjd->tj', x, rows_dn)
        gelu = jax.nn.gelu(dots, approximate=False)
        w = gate * gelu
        out = jnp.einsum('tj,tjd->td', w, rows_up)
        return out.reshape(B, N, D)
    out = _make_sc_call()(x, idx2, gate2, key_in, key_out)
    return out.reshape(B, N, D)


# speedup vs baseline: 8.1834x; 1.3018x over previous
import functools

import jax
import jax.numpy as jnp
from jax import lax
from jax.experimental import pallas as pl
from jax.experimental.pallas import tpu as pltpu
from jax.experimental.pallas import tpu_sc as plsc

B, N, D = 2, 2048, 1024
T = B * N
H = 8
KD = 128
K = 4
NUM_KEYS = 128
NUM_EXPERTS = 16384
HK = H * K
BN_EPS = 1e-5


_STATS_BLK = 256


def _stats_body(x_ref, o_ref):
    j = pl.program_id(0)

    xb = x_ref[...]

    @pl.when(j == 0)
    def _sum():
        o_ref[...] = jnp.zeros_like(o_ref)
        o_ref[0:1, :] = jnp.sum(xb, axis=0)[None, :]

    @pl.when(j == 1)
    def _sqdev():
        mean = o_ref[0:1, :] * (1.0 / float(T))
        d = xb - mean
        o_ref[1:2, :] = jnp.sum(d * d, axis=0)[None, :]


_stats_call = pl.pallas_call(
    _stats_body,
    grid=(2,),
    in_specs=[pl.BlockSpec((T, D), lambda j: (0, 0))],
    out_specs=pl.BlockSpec((8, D), lambda j: (0, 0)),
    out_shape=jax.ShapeDtypeStruct((8, D), jnp.float32),
    compiler_params=pltpu.CompilerParams(
        dimension_semantics=("arbitrary",)),
)


_TB = 512


def _topk4(s, payload):
    m_cols = s.shape[1]
    iota = lax.broadcasted_iota(jnp.int32, s.shape, 1)
    cur = s
    ss, pp = [], []
    for _ in range(K):
        m = jnp.max(cur, axis=1, keepdims=True)
        pos = jnp.min(jnp.where(cur == m, iota, m_cols), axis=1, keepdims=True)
        sel = iota == pos
        ss.append(m)
        pp.append(jnp.sum(jnp.where(sel, payload, 0), axis=1, keepdims=True))
        cur = jnp.where(sel, -jnp.inf, cur)
    return jnp.concatenate(ss, axis=1), jnp.concatenate(pp, axis=1)


def _route_body(x_ref, stats_ref, g_ref, b_ref, wq_ref, keys_ref,
                idx_ref, gate_ref):
    x = x_ref[...]
    inv_cnt = 1.0 / float(T)
    mean = stats_ref[0:1, :] * inv_cnt
    var = stats_ref[1:2, :] * inv_cnt
    xn = (x - mean) / jnp.sqrt(var + BN_EPS) * g_ref[...] + b_ref[...]
    q = lax.dot_general(xn, wq_ref[...], (((1,), (0,)), ((), ())),
                        precision=lax.Precision.DEFAULT,
                        preferred_element_type=jnp.float32)
    q = q.astype(jnp.bfloat16)
    idx_cols, gate_cols = [], []
    for h in range(H):
        sims = []
        for p in range(2):
            off = (p * H + h) * KD
            qs = q[:, off:off + KD]
            km = keys_ref[p, h].astype(jnp.bfloat16)
            sims.append(lax.dot_general(
                qs, km, (((1,), (0,)), ((), ())),
                precision=lax.Precision.DEFAULT,
                preferred_element_type=jnp.float32))
        iota_k = lax.broadcasted_iota(jnp.int32, sims[0].shape, 1)
        sx, ix = _topk4(sims[0], iota_k)
        sy, iy = _topk4(sims[1], iota_k)
        cols_s, cols_i = [], []
        for a in range(K):
            for c in range(K):
                cols_s.append(sx[:, a:a + 1] + sy[:, c:c + 1])
                cols_i.append(ix[:, a:a + 1] * NUM_KEYS + iy[:, c:c + 1])
        s16 = jnp.concatenate(cols_s, axis=1)
        i16 = jnp.concatenate(cols_i, axis=1)
        sc, ei = _topk4(s16, i16)
        m = jnp.max(sc, axis=1, keepdims=True)
        e = jnp.exp(sc - m)
        gate_cols.append(e / jnp.sum(e, axis=1, keepdims=True))
        idx_cols.append(ei)
    idx_ref[...] = jnp.concatenate(idx_cols, axis=1)
    gate_ref[...] = jnp.concatenate(gate_cols, axis=1)


_route_call = pl.pallas_call(
    _route_body,
    grid=(T // _TB,),
    in_specs=[
        pl.BlockSpec((_TB, D), lambda i: (i, 0)),
        pl.BlockSpec((8, D), lambda i: (0, 0)),
        pl.BlockSpec((1, D), lambda i: (0, 0)),
        pl.BlockSpec((1, D), lambda i: (0, 0)),
        pl.BlockSpec((D, 2 * H * KD), lambda i: (0, 0)),
        pl.BlockSpec((2, H, KD, NUM_KEYS), lambda i: (0, 0, 0, 0)),
    ],
    out_specs=[
        pl.BlockSpec((_TB, HK), lambda i: (i, 0)),
        pl.BlockSpec((_TB, HK), lambda i: (i, 0)),
    ],
    out_shape=[
        jax.ShapeDtypeStruct((T, HK), jnp.int32),
        jax.ShapeDtypeStruct((T, HK), jnp.float32),
    ],
)


_NC, _NS = 2, 16
_NW = _NC * _NS
_TPW = T // _NW
_G = 8
_UPG = 2 * _G
_L = 16


def _sc_body(x_hbm, idx_hbm, gate_hbm, kin_hbm, kout_hbm, out_hbm,
             xg, idxg, gateg, rdn, rup, outg,
             sem_dn0, sem_dn1, sem_up0, sem_up1):
    wid = lax.axis_index("s") * _NC + lax.axis_index("c")
    t0 = wid * _TPW
    jiota = lax.iota(jnp.int32, _L)
    sems = ((sem_dn0, sem_up0), (sem_dn1, sem_up1))

    def issue(u, par):
        sd, su = sems[par]
        pltpu.async_copy(kin_hbm.at[idxg.at[u]], rdn.at[par], sd)
        pltpu.async_copy(kout_hbm.at[idxg.at[u]], rup.at[par], su)

    def wait_dn(u, par):
        sd, _ = sems[par]
        pltpu.make_async_copy(kin_hbm.at[idxg.at[u]], rdn.at[par], sd).wait()

    def wait_up(u, par):
        _, su = sems[par]
        pltpu.make_async_copy(kout_hbm.at[idxg.at[u]], rup.at[par], su).wait()

    def compute(u, t_loc, half, par):
        dn = rdn.at[par]
        up = rup.at[par]
        wait_dn(u, par)
        xrow = xg.at[t_loc]

        def dot_step(c, accs):
            base = c * _L
            xv = xrow[pl.ds(base, _L)]
            new = list(accs)
            for j in range(_L):
                rv = dn[j, pl.ds(base, _L)]
                new[j] = new[j] + rv * xv
            return tuple(new)

        accs = lax.fori_loop(
            0, D // _L, dot_step,
            tuple(jnp.zeros((_L,), jnp.float32) for _ in range(_L)))
        dots = jnp.zeros((_L,), jnp.float32)
        for j in range(_L):
            pal = accs[j] + lax.rev(accs[j], (0,))
            dj = (((pal[0] + pal[1]) + (pal[2] + pal[3]))
                  + ((pal[4] + pal[5]) + (pal[6] + pal[7])))
            dots = jnp.where(jiota == j, dj, dots)
        y = dots * 0.7071067811865476
        ay = jnp.abs(y)
        tt = 1.0 / (1.0 + 0.3275911 * ay)
        poly = ((((1.061405429 * tt - 1.453152027) * tt + 1.421413741)
                 * tt - 0.284496736) * tt + 0.254829592) * tt
        erf_a = 1.0 - poly * jnp.exp(-(ay * ay))
        erf_y = jnp.where(y >= 0.0, erf_a, -erf_a)
        gelu = 0.5 * dots * (1.0 + erf_y)
        w = gateg.at[u][...] * gelu
        wait_up(u, par)
        ws = [w[j] for j in range(_L)]
        orow = outg.at[t_loc]
        is_first = half == 0

        def up_step(c, c3):
            base = c * _L
            terms = [ws[j] * up[j, pl.ds(base, _L)] for j in range(_L)]
            while len(terms) > 1:
                terms = [terms[i] + terms[i + 1]
                         for i in range(0, len(terms), 2)]
            prev = orow[pl.ds(base, _L)]
            orow[pl.ds(base, _L)] = jnp.where(is_first, terms[0],
                                              prev + terms[0])
            return c3

        lax.fori_loop(0, D // _L, up_step, 0)

    def group_body(g, carry):
        tg = t0 + g * _G
        pltpu.sync_copy(x_hbm.at[pl.ds(tg, _G)], xg)
        pltpu.sync_copy(idx_hbm.at[pl.ds(2 * tg, _UPG)], idxg)
        pltpu.sync_copy(gate_hbm.at[pl.ds(2 * tg, _UPG)], gateg)
        issue(0, 0)

        def tok_body(t_loc, c2):
            u0 = 2 * t_loc
            issue(u0 + 1, 1)
            compute(u0, t_loc, 0, 0)

            @pl.when(t_loc < _G - 1)
            def _pref():
                issue(u0 + 2, 0)

            compute(u0 + 1, t_loc, 1, 1)
            return c2

        lax.fori_loop(0, _G, tok_body, 0)
        pltpu.sync_copy(outg, out_hbm.at[pl.ds(tg, _G)])
        return carry

    lax.fori_loop(0, _TPW // _G, group_body, 0)


def _make_sc_call():
    return functools.partial(
        pl.kernel,
        out_type=jax.ShapeDtypeStruct((T, D), jnp.float32),
        mesh=plsc.VectorSubcoreMesh(core_axis_name="c", subcore_axis_name="s"),
        compiler_params=pltpu.CompilerParams(use_tc_tiling_on_sc=False),
        scratch_types=[
            pltpu.VMEM((_G, D), jnp.float32),
            pltpu.VMEM((_UPG, _L), jnp.int32),
            pltpu.VMEM((_UPG, _L), jnp.float32),
            pltpu.VMEM((2, _L, D), jnp.float32),
            pltpu.VMEM((2, _L, D), jnp.float32),
            pltpu.VMEM((_G, D), jnp.float32),
            pltpu.SemaphoreType.DMA,
            pltpu.SemaphoreType.DMA,
            pltpu.SemaphoreType.DMA,
            pltpu.SemaphoreType.DMA,
        ],
    )(_sc_body)


def kernel(inputs, bn_gamma, bn_beta, w_q, keys, key_in, key_out):
    x = inputs.reshape(T, D)
    stats = _stats_call(x)
    keys_r = jnp.transpose(keys, (2, 0, 3, 1))
    idx, gate = _route_call(x, stats, bn_gamma.reshape(1, D),
                            bn_beta.reshape(1, D),
                            w_q.astype(jnp.bfloat16), keys_r)
    idx2 = idx.reshape(2 * T, _L)
    gate2 = gate.reshape(2 * T, _L)
    out = _make_sc_call()(x, idx2, gate2, key_in, key_out)
    return out.reshape(B, N, D)


# speedup vs baseline: 9.5793x; 1.1706x over previous
import functools

import jax
import jax.numpy as jnp
from jax import lax
from jax.experimental import pallas as pl
from jax.experimental.pallas import tpu as pltpu
from jax.experimental.pallas import tpu_sc as plsc

B, N, D = 2, 2048, 1024
T = B * N
H = 8
KD = 128
K = 4
NUM_KEYS = 128
NUM_EXPERTS = 16384
HK = H * K
BN_EPS = 1e-5


_STATS_BLK = 256


def _stats_body(x_ref, o_ref):
    j = pl.program_id(0)

    xb = x_ref[...]

    @pl.when(j == 0)
    def _sum():
        o_ref[...] = jnp.zeros_like(o_ref)
        o_ref[0:1, :] = jnp.sum(xb, axis=0)[None, :]

    @pl.when(j == 1)
    def _sqdev():
        mean = o_ref[0:1, :] * (1.0 / float(T))
        d = xb - mean
        o_ref[1:2, :] = jnp.sum(d * d, axis=0)[None, :]


_stats_call = pl.pallas_call(
    _stats_body,
    grid=(2,),
    in_specs=[pl.BlockSpec((T, D), lambda j: (0, 0))],
    out_specs=pl.BlockSpec((8, D), lambda j: (0, 0)),
    out_shape=jax.ShapeDtypeStruct((8, D), jnp.float32),
    compiler_params=pltpu.CompilerParams(
        dimension_semantics=("arbitrary",)),
)


_TB = 512


def _topk4(s, payload):
    m_cols = s.shape[1]
    iota = lax.broadcasted_iota(jnp.int32, s.shape, 1)
    cur = s
    ss, pp = [], []
    for _ in range(K):
        m = jnp.max(cur, axis=1, keepdims=True)
        pos = jnp.min(jnp.where(cur == m, iota, m_cols), axis=1, keepdims=True)
        sel = iota == pos
        ss.append(m)
        pp.append(jnp.sum(jnp.where(sel, payload, 0), axis=1, keepdims=True))
        cur = jnp.where(sel, -jnp.inf, cur)
    return jnp.concatenate(ss, axis=1), jnp.concatenate(pp, axis=1)


def _route_body(x_ref, stats_ref, g_ref, b_ref, wq_ref, keys_ref,
                idx_ref, gate_ref):
    x = x_ref[...]
    inv_cnt = 1.0 / float(T)
    mean = stats_ref[0:1, :] * inv_cnt
    var = stats_ref[1:2, :] * inv_cnt
    xn = (x - mean) / jnp.sqrt(var + BN_EPS) * g_ref[...] + b_ref[...]
    q = lax.dot_general(xn, wq_ref[...], (((1,), (0,)), ((), ())),
                        precision=lax.Precision.DEFAULT,
                        preferred_element_type=jnp.float32)
    q = q.astype(jnp.bfloat16)
    idx_cols, gate_cols = [], []
    for h in range(H):
        sims = []
        for p in range(2):
            off = (p * H + h) * KD
            qs = q[:, off:off + KD]
            km = keys_ref[p, h].astype(jnp.bfloat16)
            sims.append(lax.dot_general(
                qs, km, (((1,), (0,)), ((), ())),
                precision=lax.Precision.DEFAULT,
                preferred_element_type=jnp.float32))
        iota_k = lax.broadcasted_iota(jnp.int32, sims[0].shape, 1)
        sx, ix = _topk4(sims[0], iota_k)
        sy, iy = _topk4(sims[1], iota_k)
        cols_s, cols_i = [], []
        for a in range(K):
            for c in range(K):
                cols_s.append(sx[:, a:a + 1] + sy[:, c:c + 1])
                cols_i.append(ix[:, a:a + 1] * NUM_KEYS + iy[:, c:c + 1])
        s16 = jnp.concatenate(cols_s, axis=1)
        i16 = jnp.concatenate(cols_i, axis=1)
        sc, ei = _topk4(s16, i16)
        m = jnp.max(sc, axis=1, keepdims=True)
        e = jnp.exp(sc - m)
        gate_cols.append(e / jnp.sum(e, axis=1, keepdims=True))
        idx_cols.append(ei)
    idx_ref[...] = jnp.concatenate(idx_cols, axis=1)
    gate_ref[...] = jnp.concatenate(gate_cols, axis=1)


def _make_route_call(nt):
    return pl.pallas_call(
        _route_body,
        grid=(nt // _TB,),
        in_specs=[
            pl.BlockSpec((_TB, D), lambda i: (i, 0)),
            pl.BlockSpec((8, D), lambda i: (0, 0)),
            pl.BlockSpec((1, D), lambda i: (0, 0)),
            pl.BlockSpec((1, D), lambda i: (0, 0)),
            pl.BlockSpec((D, 2 * H * KD), lambda i: (0, 0)),
            pl.BlockSpec((2, H, KD, NUM_KEYS), lambda i: (0, 0, 0, 0)),
        ],
        out_specs=[
            pl.BlockSpec((_TB, HK), lambda i: (i, 0)),
            pl.BlockSpec((_TB, HK), lambda i: (i, 0)),
        ],
        out_shape=[
            jax.ShapeDtypeStruct((nt, HK), jnp.int32),
            jax.ShapeDtypeStruct((nt, HK), jnp.float32),
        ],
    )


_NC, _NS = 2, 16
_NW = _NC * _NS
_TPW = T // _NW
_G = 8
_UPG = 2 * _G
_L = 16


def _make_sc_body(tpw):
  def _sc_body(x_hbm, idx_hbm, gate_hbm, kin_hbm, kout_hbm, out_hbm,
               xg, idxg, gateg, rdn, rup, outg,
               sem_dn0, sem_dn1, sem_up0, sem_up1):
    wid = lax.axis_index("s") * _NC + lax.axis_index("c")
    t0 = wid * tpw
    jiota = lax.iota(jnp.int32, _L)
    sems = ((sem_dn0, sem_up0), (sem_dn1, sem_up1))

    def issue(u, par):
        sd, su = sems[par]
        pltpu.async_copy(kin_hbm.at[idxg.at[u]], rdn.at[par], sd)
        pltpu.async_copy(kout_hbm.at[idxg.at[u]], rup.at[par], su)

    def wait_dn(u, par):
        sd, _ = sems[par]
        pltpu.make_async_copy(kin_hbm.at[idxg.at[u]], rdn.at[par], sd).wait()

    def wait_up(u, par):
        _, su = sems[par]
        pltpu.make_async_copy(kout_hbm.at[idxg.at[u]], rup.at[par], su).wait()

    def compute(u, t_loc, half, par):
        dn = rdn.at[par]
        up = rup.at[par]
        wait_dn(u, par)
        xrow = xg.at[t_loc]

        def dot_step(c, accs):
            base = c * _L
            xv = xrow[pl.ds(base, _L)]
            new = list(accs)
            for j in range(_L):
                rv = dn[j, pl.ds(base, _L)]
                new[j] = new[j] + rv * xv
            return tuple(new)

        accs = lax.fori_loop(
            0, D // _L, dot_step,
            tuple(jnp.zeros((_L,), jnp.float32) for _ in range(_L)))
        dots = jnp.zeros((_L,), jnp.float32)
        for j in range(_L):
            pal = accs[j] + lax.rev(accs[j], (0,))
            dj = (((pal[0] + pal[1]) + (pal[2] + pal[3]))
                  + ((pal[4] + pal[5]) + (pal[6] + pal[7])))
            dots = jnp.where(jiota == j, dj, dots)
        y = dots * 0.7071067811865476
        ay = jnp.abs(y)
        tt = 1.0 / (1.0 + 0.3275911 * ay)
        poly = ((((1.061405429 * tt - 1.453152027) * tt + 1.421413741)
                 * tt - 0.284496736) * tt + 0.254829592) * tt
        erf_a = 1.0 - poly * jnp.exp(-(ay * ay))
        erf_y = jnp.where(y >= 0.0, erf_a, -erf_a)
        gelu = 0.5 * dots * (1.0 + erf_y)
        w = gateg.at[u][...] * gelu
        wait_up(u, par)
        ws = [w[j] for j in range(_L)]
        orow = outg.at[t_loc]
        is_first = half == 0

        def up_step(c, c3):
            base = c * _L
            terms = [ws[j] * up[j, pl.ds(base, _L)] for j in range(_L)]
            while len(terms) > 1:
                terms = [terms[i] + terms[i + 1]
                         for i in range(0, len(terms), 2)]
            prev = orow[pl.ds(base, _L)]
            orow[pl.ds(base, _L)] = jnp.where(is_first, terms[0],
                                              prev + terms[0])
            return c3

        lax.fori_loop(0, D // _L, up_step, 0)

    def group_body(g, carry):
        tg = t0 + g * _G
        pltpu.sync_copy(x_hbm.at[pl.ds(tg, _G)], xg)
        pltpu.sync_copy(idx_hbm.at[pl.ds(2 * tg, _UPG)], idxg)
        pltpu.sync_copy(gate_hbm.at[pl.ds(2 * tg, _UPG)], gateg)
        issue(0, 0)

        def tok_body(t_loc, c2):
            u0 = 2 * t_loc
            issue(u0 + 1, 1)
            compute(u0, t_loc, 0, 0)

            @pl.when(t_loc < _G - 1)
            def _pref():
                issue(u0 + 2, 0)

            compute(u0 + 1, t_loc, 1, 1)
            return c2

        lax.fori_loop(0, _G, tok_body, 0)
        pltpu.sync_copy(outg, out_hbm.at[pl.ds(tg, _G)])
        return carry

    lax.fori_loop(0, tpw // _G, group_body, 0)

  return _sc_body


def _make_sc_call(nt):
    return functools.partial(
        pl.kernel,
        out_type=jax.ShapeDtypeStruct((nt, D), jnp.float32),
        mesh=plsc.VectorSubcoreMesh(core_axis_name="c", subcore_axis_name="s"),
        compiler_params=pltpu.CompilerParams(use_tc_tiling_on_sc=False),
        scratch_types=[
            pltpu.VMEM((_G, D), jnp.float32),
            pltpu.VMEM((_UPG, _L), jnp.int32),
            pltpu.VMEM((_UPG, _L), jnp.float32),
            pltpu.VMEM((2, _L, D), jnp.float32),
            pltpu.VMEM((2, _L, D), jnp.float32),
            pltpu.VMEM((_G, D), jnp.float32),
            pltpu.SemaphoreType.DMA,
            pltpu.SemaphoreType.DMA,
            pltpu.SemaphoreType.DMA,
            pltpu.SemaphoreType.DMA,
        ],
    )(_make_sc_body(nt // _NW))


_NCHUNK = 2


def kernel(inputs, bn_gamma, bn_beta, w_q, keys, key_in, key_out):
    x = inputs.reshape(T, D)
    stats = _stats_call(x)
    keys_r = jnp.transpose(keys, (2, 0, 3, 1))
    wq16 = w_q.astype(jnp.bfloat16)
    gamma = bn_gamma.reshape(1, D)
    beta = bn_beta.reshape(1, D)
    nt = T // _NCHUNK
    route = _make_route_call(nt)
    sc = _make_sc_call(nt)
    outs = []
    for ci in range(_NCHUNK):
        xc = lax.slice(x, (ci * nt, 0), ((ci + 1) * nt, D))
        idx, gate = route(xc, stats, gamma, beta, wq16, keys_r)
        idx2 = idx.reshape(2 * nt, _L)
        gate2 = gate.reshape(2 * nt, _L)
        outs.append(sc(xc, idx2, gate2, key_in, key_out))
    out = jnp.concatenate(outs, axis=0)
    return out.reshape(B, N, D)
